# baseline (speedup 1.0000x reference)
import functools

import jax
import jax.numpy as jnp
from jax import lax
from jax.experimental import pallas as pl
from jax.experimental.pallas import tpu as pltpu
from jax.experimental.pallas import tpu_sc as plsc

OUT_DIM = 512
NC, NS = 2, 16
NW = NC * NS
CHUNK = 32
NBUF = 4


def _table_body(sym_ref, w_ref, b_ref, out_ref):
    t = (
        jnp.dot(sym_ref[:], w_ref[:], preferred_element_type=jnp.float32)
        + b_ref[:]
    )
    out_ref[:] = jnp.broadcast_to(t[None], (NW,) + t.shape)


def _make_table(symbols, W, b):
    vocab = symbols.shape[0]
    rep = pl.pallas_call(
        _table_body,
        out_shape=jax.ShapeDtypeStruct((NW, vocab, OUT_DIM), jnp.float32),
    )(symbols, W, b.reshape(1, OUT_DIM))
    return rep.reshape(NW * vocab, OUT_DIM)


def _gather_body(n_chunks, table, idx, out, idx_v, *scratch):
    bufs = scratch[:NBUF]
    gsems = scratch[NBUF:2 * NBUF]
    wsems = scratch[2 * NBUF:3 * NBUF]
    wid = lax.axis_index("s") * NC + lax.axis_index("c")
    base = wid * (n_chunks * CHUNK)
    pltpu.sync_copy(idx.at[wid], idx_v)

    def gstart(c, b):
        pltpu.async_copy(table.at[idx_v.at[pl.ds(c * CHUNK, CHUNK)]],
                         bufs[b], gsems[b])

    def gwait(b):
        pltpu.make_async_copy(table.at[idx_v.at[pl.ds(0, CHUNK)]],
                              bufs[b], gsems[b]).wait()

    def wstart(c, b):
        pltpu.async_copy(bufs[b], out.at[pl.ds(base + c * CHUNK, CHUNK)],
                         wsems[b])

    def wwait(c, b):
        pltpu.make_async_copy(
            bufs[b], out.at[pl.ds(base + c * CHUNK, CHUNK)], wsems[b]).wait()

    for b in range(NBUF):
        gstart(b, b)

    def body(i, carry):
        j = NBUF * i
        for b in range(NBUF):
            gwait(b)
            wstart(j + b, b)
        for b in range(NBUF):
            wwait(j + b, b)
            gstart(j + b + NBUF, b)
        return carry

    lax.fori_loop(0, n_chunks // NBUF - 1, body, 0)

    j = n_chunks - NBUF
    for b in range(NBUF):
        gwait(b)
        wstart(j + b, b)
    for b in range(NBUF):
        wwait(j + b, b)


def _gather_rows(table, idx2d, n_chunks):
    rows = NW * n_chunks * CHUNK
    mesh = plsc.VectorSubcoreMesh(core_axis_name="c", subcore_axis_name="s")
    k = pl.kernel(
        functools.partial(_gather_body, n_chunks),
        mesh=mesh,
        out_type=jax.ShapeDtypeStruct((rows, OUT_DIM), jnp.float32),
        scratch_types=(
            [pltpu.VMEM((n_chunks * CHUNK,), jnp.int32)]
            + [pltpu.VMEM((CHUNK, OUT_DIM), jnp.float32)] * NBUF
            + [pltpu.SemaphoreType.DMA] * (2 * NBUF)
        ),
        compiler_params=pltpu.CompilerParams(
            needs_layout_passes=False, use_tc_tiling_on_sc=False),
    )
    return k(table, idx2d)


def kernel(QR, symbols, W, b):
    batch, seq = QR.shape
    rows = batch * seq
    n_chunks = rows // (NW * CHUNK)
    assert rows == NW * n_chunks * CHUNK and n_chunks % NBUF == 0

    vocab = symbols.shape[0]
    table = _make_table(symbols, W, b)
    idx2d = QR.astype(jnp.int32).reshape(NW, n_chunks * CHUNK)
    idx2d = idx2d + (jnp.arange(NW, dtype=jnp.int32) * vocab)[:, None]
    out = _gather_rows(table, idx2d, n_chunks)
    return out.reshape(batch, seq, OUT_DIM)


# speedup vs baseline: 2.2350x; 2.2350x over previous
import functools

import jax
import jax.numpy as jnp
from jax import lax
from jax.experimental import pallas as pl
from jax.experimental.pallas import tpu as pltpu
from jax.experimental.pallas import tpu_sc as plsc

OUT_DIM = 512
NC, NS = 2, 16
NW = NC * NS
CHUNK = 16
NBUF = 8


def _table_body(sym_ref, w_ref, b_ref, out_ref):
    t = (
        jnp.dot(sym_ref[:], w_ref[:], preferred_element_type=jnp.float32)
        + b_ref[:]
    )
    out_ref[:] = jnp.broadcast_to(t[None], (NW,) + t.shape)


def _make_table(symbols, W, b):
    vocab = symbols.shape[0]
    rep = pl.pallas_call(
        _table_body,
        out_shape=jax.ShapeDtypeStruct((NW, vocab, OUT_DIM), jnp.float32),
    )(symbols, W, b.reshape(1, OUT_DIM))
    return rep.reshape(NW * vocab, OUT_DIM)


def _gather_body(n_chunks, table, idx, out, idx_v, *scratch):
    bufs = scratch[:NBUF]
    gsems = scratch[NBUF:2 * NBUF]
    wsems = scratch[2 * NBUF:3 * NBUF]
    wid = lax.axis_index("s") * NC + lax.axis_index("c")
    base = wid * (n_chunks * CHUNK)
    pltpu.sync_copy(idx.at[wid], idx_v)

    def gstart(c, b):
        pltpu.async_copy(table.at[idx_v.at[pl.ds(c * CHUNK, CHUNK)]],
                         bufs[b], gsems[b])

    def gwait(b):
        pltpu.make_async_copy(table.at[idx_v.at[pl.ds(0, CHUNK)]],
                              bufs[b], gsems[b]).wait()

    def wstart(c, b):
        pltpu.async_copy(bufs[b], out.at[pl.ds(base + c * CHUNK, CHUNK)],
                         wsems[b])

    def wwait(c, b):
        pltpu.make_async_copy(
            bufs[b], out.at[pl.ds(base + c * CHUNK, CHUNK)], wsems[b]).wait()

    for b in range(NBUF):
        gstart(b, b)

    def body(i, carry):
        j = NBUF * i
        for b in range(NBUF):
            gwait(b)
            wstart(j + b, b)
        for b in range(NBUF):
            wwait(j + b, b)
            gstart(j + b + NBUF, b)
        return carry

    lax.fori_loop(0, n_chunks // NBUF - 1, body, 0)

    j = n_chunks - NBUF
    for b in range(NBUF):
        gwait(b)
        wstart(j + b, b)
    for b in range(NBUF):
        wwait(j + b, b)


def _gather_rows(table, idx2d, n_chunks):
    rows = NW * n_chunks * CHUNK
    mesh = plsc.VectorSubcoreMesh(core_axis_name="c", subcore_axis_name="s")
    k = pl.kernel(
        functools.partial(_gather_body, n_chunks),
        mesh=mesh,
        out_type=jax.ShapeDtypeStruct((rows, OUT_DIM), jnp.float32),
        scratch_types=(
            [pltpu.VMEM((n_chunks * CHUNK,), jnp.int32)]
            + [pltpu.VMEM((CHUNK, OUT_DIM), jnp.float32)] * NBUF
            + [pltpu.SemaphoreType.DMA] * (2 * NBUF)
        ),
        compiler_params=pltpu.CompilerParams(needs_layout_passes=False),
    )
    return k(table, idx2d)


def kernel(QR, symbols, W, b):
    batch, seq = QR.shape
    rows = batch * seq
    n_chunks = rows // (NW * CHUNK)
    assert rows == NW * n_chunks * CHUNK and n_chunks % NBUF == 0

    vocab = symbols.shape[0]
    table = _make_table(symbols, W, b)
    idx2d = QR.astype(jnp.int32).reshape(NW, n_chunks * CHUNK)
    idx2d = idx2d + (jnp.arange(NW, dtype=jnp.int32) * vocab)[:, None]
    out = _gather_rows(table, idx2d, n_chunks)
    return out.reshape(batch, seq, OUT_DIM)
